# Initial kernel scaffold; baseline (speedup 1.0000x reference)
import jax
import jax.numpy as jnp
from jax.experimental import pallas as pl


def kernel(x, ln_scale, ln_bias, W1, b1, W2, b2):
    raise NotImplementedError("write your pallas kernel here")


# speedup vs baseline: 13.4352x; 13.4352x over previous
import functools

import jax
import jax.numpy as jnp
from jax.experimental import pallas as pl
from jax.experimental.pallas import tpu as pltpu

_K = 20
_N = 1024
_D = 64
_H = 128
_BIG = 1e10
_EDIM = 2 * _D + 1


def _edgeconv_kernel(xb_ref, w1x_ref, w1d_ref, w1e_ref, csum_ref, b1_ref,
                     w2_ref, b2_ref, out_ref, d_ref, ax_ref):
    xb = xb_ref[0]

    g = jax.lax.dot_general(xb, xb, (((1,), (1,)), ((), ())),
                            preferred_element_type=jnp.float32,
                            precision=jax.lax.Precision.HIGHEST)
    colid = jax.lax.broadcasted_iota(jnp.int32, (_N, _N), 1)
    rowid = jax.lax.broadcasted_iota(jnp.int32, (_N, _N), 0)
    eye = colid == rowid
    sqj = jnp.sum(jnp.where(eye, g, 0.0), axis=0, keepdims=True)
    d_ref[...] = sqj - 2.0 * g + jnp.where(eye, _BIG, 0.0)

    ax_ref[...] = jax.lax.dot_general(xb, w1x_ref[...], (((1,), (0,)), ((), ())),
                                      preferred_element_type=jnp.float32)
    out_ref[0] = jnp.full((_N, _D), -jnp.inf, dtype=jnp.float32)

    sx = jnp.sum(xb, axis=1, keepdims=True)
    ssx = jnp.sum(xb * xb, axis=1, keepdims=True)

    def body(t, carry):
        d = d_ref[...]
        mn = jnp.min(d, axis=1, keepdims=True)
        idxsel = jnp.min(jnp.where(d <= mn, colid, jnp.int32(2 ** 30)),
                         axis=1, keepdims=True)
        onehot_b = colid == idxsel
        d_ref[...] = jnp.where(onehot_b, jnp.float32(3e38), d)
        oh = onehot_b.astype(jnp.float32)
        nbr = jax.lax.dot_general(oh, xb, (((1,), (0,)), ((), ())),
                                  preferred_element_type=jnp.float32)
        diff = nbr - xb
        e = jnp.sum(diff * diff, axis=1, keepdims=True)
        mean = (sx + jnp.sum(diff, axis=1, keepdims=True) + e) * (1.0 / _EDIM)
        msq = (ssx + e + e * e) * (1.0 / _EDIM)
        r = jax.lax.rsqrt(msq - mean * mean + 1e-6)
        ad = jax.lax.dot_general(diff, w1d_ref[...], (((1,), (0,)), ((), ())),
                                 preferred_element_type=jnp.float32)
        h1 = (r * (ax_ref[...] + ad + e * w1e_ref[...])
              - (mean * r) * csum_ref[...] + b1_ref[...])
        h = h1 * (1.0 / (1.0 + jnp.exp(-h1)))
        h2 = jax.lax.dot_general(h, w2_ref[...], (((1,), (0,)), ((), ())),
                                 preferred_element_type=jnp.float32) + b2_ref[...]
        out_ref[0] = jnp.maximum(out_ref[0], h2)
        return carry

    jax.lax.fori_loop(0, _K, body, 0)


@jax.jit
def kernel(x, ln_scale, ln_bias, W1, b1, W2, b2):
    B, N, D = x.shape
    w1s = ln_scale[:, None] * W1
    b1f = (b1 + ln_bias @ W1)[None, :]
    csum = jnp.sum(w1s, axis=0)[None, :]
    w1x = w1s[:D]
    w1d = w1s[D:2 * D]
    w1e = w1s[2 * D:2 * D + 1]

    grid = (B,)
    out = pl.pallas_call(
        _edgeconv_kernel,
        grid=grid,
        in_specs=[
            pl.BlockSpec((1, N, D), lambda b: (b, 0, 0)),
            pl.BlockSpec((D, _H), lambda b: (0, 0)),
            pl.BlockSpec((D, _H), lambda b: (0, 0)),
            pl.BlockSpec((1, _H), lambda b: (0, 0)),
            pl.BlockSpec((1, _H), lambda b: (0, 0)),
            pl.BlockSpec((1, _H), lambda b: (0, 0)),
            pl.BlockSpec((_H, D), lambda b: (0, 0)),
            pl.BlockSpec((1, D), lambda b: (0, 0)),
        ],
        out_specs=pl.BlockSpec((1, N, D), lambda b: (b, 0, 0)),
        out_shape=jax.ShapeDtypeStruct((B, N, D), jnp.float32),
        scratch_shapes=[
            pltpu.VMEM((_N, _N), jnp.float32),
            pltpu.VMEM((_N, _H), jnp.float32),
        ],
    )(x, w1x, w1d, w1e, csum, b1f, W2, b2[None, :])
    return out
